# baseline (speedup 1.0000x reference)
import functools

import jax
import jax.numpy as jnp
import numpy as np
from jax import lax
from jax.experimental import pallas as pl
from jax.experimental.pallas import tpu as pltpu
from jax.experimental.pallas import tpu_sc as plsc

N = 10000
E = 320000
H = 128
CUTOFF = 10.0
NUM_CONVS = 3

NCORE = 2
NSUB = 16
NW = NCORE * NSUB
EW = E // NW
K = 80
NCH = EW // K
N_PAD = 10240
ROWS_PER_TILE = N_PAD // NSUB
LANES = 16
HW = H // 2
NBLK = H // 32

_PERM = np.concatenate([
    np.stack([np.arange(16), np.arange(16) + 16], axis=1).reshape(-1) + 32 * b
    for b in range(NBLK)
])


def _emb_kernel(z_ref, w0_ref, b0_ref, w1_ref, b1_ref, w1p_ref, b1p_ref,
                o_ref, os_ref):
    h = jnp.maximum(
        jnp.dot(z_ref[...], w0_ref[...], preferred_element_type=jnp.float32)
        + b0_ref[...], 0.0)
    o_ref[...] = jnp.dot(h, w1_ref[...],
                         preferred_element_type=jnp.float32) + b1_ref[...]
    os_ref[...] = jnp.dot(h, w1p_ref[...],
                          preferred_element_type=jnp.float32) + b1p_ref[...]


def _edge_kernel(ea_ref, el_ref, w0_ref, b0_ref, w1p_ref, b1p_ref, o_ref):
    h = jnp.maximum(
        jnp.dot(ea_ref[...], w0_ref[...], preferred_element_type=jnp.float32)
        + b0_ref[...], 0.0)
    y = jnp.dot(h, w1p_ref[...],
                preferred_element_type=jnp.float32) + b1p_ref[...]
    o_ref[...] = jnp.where(el_ref[...] <= CUTOFF, y, 0.0).astype(jnp.bfloat16)


def _update_kernel(parts_ref0, parts_ref1, cv_ref, cvs_ref, w0_ref, b0_ref,
                   w1_ref, b1_ref, w1p_ref, b1p_ref, o_ref, os_ref, *,
                   apply_relu):
    cv = cv_ref[...]
    out = parts_ref0[0] + parts_ref1[0] + cv
    h = jnp.maximum(
        jnp.dot(out, w0_ref[...], preferred_element_type=jnp.float32)
        + b0_ref[...], 0.0)
    y = jnp.dot(h, w1_ref[...],
                preferred_element_type=jnp.float32) + b1_ref[...]
    ys = jnp.dot(h, w1p_ref[...],
                 preferred_element_type=jnp.float32) + b1p_ref[...]
    if apply_relu:
        y = jnp.maximum(y, 0.0)
        ys = jnp.maximum(ys, 0.0)
    o_ref[...] = y + cv
    os_ref[...] = ys + cvs_ref[...]


def _full_spec(shape):
    return pl.BlockSpec(shape, lambda i: (0,) * len(shape))


def _emb(z, w0, b0, w1, b1, w1p, b1p):
    bn = 2000
    return pl.pallas_call(
        _emb_kernel,
        grid=(N // bn,),
        in_specs=[
            pl.BlockSpec((bn, z.shape[1]), lambda i: (i, 0)),
            _full_spec(w0.shape), _full_spec(b0.shape),
            _full_spec(w1.shape), _full_spec(b1.shape),
            _full_spec(w1p.shape), _full_spec(b1p.shape),
        ],
        out_specs=[pl.BlockSpec((bn, H), lambda i: (i, 0))] * 2,
        out_shape=[jax.ShapeDtypeStruct((N, H), jnp.float32)] * 2,
    )(z, w0, b0, w1, b1, w1p, b1p)


def _edge_mlp(ea, el, w0, b0, w1p, b1p):
    be = 4000
    return pl.pallas_call(
        _edge_kernel,
        grid=(E // be,),
        in_specs=[
            pl.BlockSpec((be, H), lambda i: (i, 0)),
            pl.BlockSpec((be, 1), lambda i: (i, 0)),
            _full_spec(w0.shape), _full_spec(b0.shape),
            _full_spec(w1p.shape), _full_spec(b1p.shape),
        ],
        out_specs=pl.BlockSpec((be, H), lambda i: (i, 0)),
        out_shape=jax.ShapeDtypeStruct((E, H), jnp.bfloat16),
    )(ea, el, w0, b0, w1p, b1p)


def _update(parts, cv, cvs, w0, b0, w1, b1, w1p, b1p, apply_relu):
    bn = 2000
    return pl.pallas_call(
        functools.partial(_update_kernel, apply_relu=apply_relu),
        grid=(N // bn,),
        in_specs=[
            pl.BlockSpec((1, bn, H), lambda i: (0, i, 0)),
            pl.BlockSpec((1, bn, H), lambda i: (1, i, 0)),
            pl.BlockSpec((bn, H), lambda i: (i, 0)),
            pl.BlockSpec((bn, H), lambda i: (i, 0)),
            _full_spec(w0.shape), _full_spec(b0.shape),
            _full_spec(w1.shape), _full_spec(b1.shape),
            _full_spec(w1p.shape), _full_spec(b1p.shape),
        ],
        out_specs=[pl.BlockSpec((bn, H), lambda i: (i, 0))] * 2,
        out_shape=[jax.ShapeDtypeStruct((N, H), jnp.float32)] * 2,
    )(parts, parts, cv, cvs, w0, b0, w1, b1, w1p, b1p)


@functools.cache
def _make_sc_propagate():
    mesh = plsc.VectorSubcoreMesh(core_axis_name="c", subcore_axis_name="s",
                                  num_cores=NCORE, num_subcores=NSUB)
    return pl.kernel(
        _sc_propagate_body,
        out_type=jax.ShapeDtypeStruct((NCORE, N_PAD, H), jnp.float32),
        mesh=mesh,
        compiler_params=pltpu.CompilerParams(use_tc_tiling_on_sc=False, needs_layout_passes=False),
        scratch_types=[
            pltpu.VMEM((2, K), jnp.int32),
            pltpu.VMEM((2, K), jnp.int32),
            pltpu.VMEM((2, K), jnp.int32),
            pltpu.VMEM((2, K), jnp.int32),
            pltpu.VMEM((K, HW), jnp.int32),
            pltpu.VMEM((K, HW), jnp.int32),
            pltpu.VMEM((K, HW), jnp.int32),
            pltpu.VMEM((K, HW), jnp.int32),
            pltpu.VMEM((K, H), jnp.float32),
            pltpu.VMEM((K, H), jnp.float32),
            pltpu.VMEM_SHARED((N_PAD, H), jnp.float32),
            pltpu.SemaphoreType.DMA,
            pltpu.SemaphoreType.DMA,
            pltpu.SemaphoreType.DMA,
            pltpu.SemaphoreType.DMA,
            pltpu.SemaphoreType.DMA,
            pltpu.SemaphoreType.DMA,
        ],
    )


_HIMASK = -65536


def _sc_propagate_body(x_hbm, wm_hbm, ei_hbm, out_hbm,
                       iv0, iv1, iv2, iv3, xb0, xb1, wb0, wb1, mb0, mb1,
                       acc, si0, si1, sd0, sd1, ss0, ss1):
    cid = lax.axis_index("c")
    sid = lax.axis_index("s")
    wid = cid * NSUB + sid
    IV = (iv0, iv1, iv2, iv3)
    XB, WB, MB = (xb0, xb1), (wb0, wb1), (mb0, mb1)
    SI, SD, SS = (si0, si1), (sd0, sd1), (ss0, ss1)

    @plsc.parallel_loop(0, K)
    def _zero_row(e):
        for kk in range(H // LANES):
            mb0[e, pl.ds(kk * LANES, LANES)] = jnp.zeros((LANES,), jnp.float32)
    for r in range(ROWS_PER_TILE // K):
        pltpu.sync_copy(mb0, acc.at[pl.ds(sid * ROWS_PER_TILE + r * K, K)])

    plsc.subcore_barrier()

    def _issue_idx(j, b2, b4):
        pltpu.async_copy(ei_hbm.at[wid, j], IV[b4], SI[b2])

    def _wait_idx(b2, b4):
        pltpu.make_async_copy(ei_hbm.at[wid, 0], IV[b4], SI[b2]).wait()

    def _issue_data(j, b2, b4):
        pltpu.async_copy(x_hbm.at[IV[b4].at[0]], XB[b2], SD[b2])
        pltpu.async_copy(wm_hbm.at[wid, j], WB[b2], SD[b2])

    def _wait_data(b2):
        pltpu.make_async_copy(wm_hbm.at[wid, 0], XB[b2], SD[b2]).wait()
        pltpu.make_async_copy(wm_hbm.at[wid, 0], WB[b2], SD[b2]).wait()

    def _compute(b2):
        xbuf, wbuf, mbuf = XB[b2], WB[b2], MB[b2]

        @plsc.parallel_loop(0, K, unroll=4)
        def _row(e):
            for kk in range(NBLK):
                sl = pl.ds(kk * LANES, LANES)
                xw = xbuf[e, sl]
                ww = wbuf[e, sl]
                x_lo = plsc.bitcast(xw << 16, jnp.float32)
                x_hi = plsc.bitcast(xw & _HIMASK, jnp.float32)
                w_lo = plsc.bitcast(ww << 16, jnp.float32)
                w_hi = plsc.bitcast(ww & _HIMASK, jnp.float32)
                mbuf[e, pl.ds(kk * 32, LANES)] = \
                    jnp.maximum(x_lo + w_lo, 0.0)
                mbuf[e, pl.ds(kk * 32 + LANES, LANES)] = \
                    jnp.maximum(x_hi + w_hi, 0.0)

    def _issue_scatter(b2, b4):
        pltpu.async_copy(MB[b2], acc.at[IV[b4].at[1]], SS[b2], add=True)

    def _wait_scatter(b2):
        pltpu.make_async_copy(MB[b2], acc.at[pl.ds(0, K)], SS[b2]).wait()

    _issue_idx(0, 0, 0)
    _wait_idx(0, 0)
    _issue_data(0, 0, 0)
    _issue_idx(1, 1, 1)

    def _steady(j2, _):
        for b in range(4):
            j = j2 * 4 + b
            b2, bn2, b4 = b % 2, (b + 1) % 2, b
            _wait_data(b2)
            _wait_idx(bn2, (b4 + 1) % 4)
            if b == 0:
                @pl.when(j2 > 0)
                def _():
                    _wait_scatter(bn2)
            else:
                _wait_scatter(bn2)
            _issue_data(j + 1, bn2, (b4 + 1) % 4)
            _compute(b2)
            _issue_scatter(b2, b4)
            if b == 3:
                @pl.when(j2 < (NCH - 1) // 4 - 1)
                def _():
                    _issue_idx(j + 2, b2, (b4 + 2) % 4)
            else:
                _issue_idx(j + 2, b2, (b4 + 2) % 4)
        return 0

    lax.fori_loop(0, (NCH - 1) // 4, _steady, 0)
    _wait_data(0)
    _wait_scatter(1)
    _compute(0)
    _issue_scatter(0, 0)
    _wait_scatter(0)
    plsc.subcore_barrier()

    sl = pl.ds(sid * ROWS_PER_TILE, ROWS_PER_TILE)
    pltpu.sync_copy(acc.at[sl], out_hbm.at[cid, sl])


def _as_words(a):
    return lax.bitcast_convert_type(
        a.reshape(*a.shape[:-1], HW, 2), jnp.int32)


def kernel(z, edge_index, edge_attr, edge_length,
           emb_W0, emb_b0, emb_W1, emb_b1,
           m1_W0, m1_b0, m1_W1, m1_b1,
           m2_W0, m2_b0, m2_W1, m2_b1):
    perm = jnp.asarray(_PERM)
    emb_W1p, emb_b1p = emb_W1[:, perm], emb_b1[perm]
    m1_W1p, m1_b1p = m1_W1[:, perm], m1_b1[perm]
    m2_W1p, m2_b1p = m2_W1[:, perm], m2_b1[perm]

    x, xs = _emb(z, emb_W0, emb_b0.reshape(1, H), emb_W1, emb_b1.reshape(1, H),
                 emb_W1p, emb_b1p.reshape(1, H))
    wm = _edge_mlp(edge_attr, edge_length.reshape(E, 1),
                   m2_W0, m2_b0.reshape(1, H), m2_W1p, m2_b1p.reshape(1, H))
    wm_w = _as_words(wm).reshape(NW, NCH, K, HW)
    ei = edge_index.astype(jnp.int32).reshape(2, NW, NCH, K).transpose(1, 2, 0, 3)

    conv, convs = x, xs
    for conv_id in range(NUM_CONVS):
        xw = _as_words(convs.astype(jnp.bfloat16))
        parts = _make_sc_propagate()(xw, wm_w, ei)
        conv, convs = _update(parts, conv, convs,
                              m1_W0, m1_b0.reshape(1, H),
                              m1_W1, m1_b1.reshape(1, H),
                              m1_W1p, m1_b1p.reshape(1, H),
                              apply_relu=conv_id < NUM_CONVS - 1)
    return conv


# speedup vs baseline: 1.7418x; 1.7418x over previous
import functools

import jax
import jax.numpy as jnp
from jax import lax
from jax.experimental import pallas as pl
from jax.experimental.pallas import tpu as pltpu
from jax.experimental.pallas import tpu_sc as plsc

N = 10000
E = 320000
H = 128
CUTOFF = 10.0
NUM_CONVS = 3

NCORE = 2
NSUB = 16
NW = NCORE * NSUB
EW = E // NW
K = 80
NCH = EW // K
N_PAD = 10240
ROWS_PER_TILE = N_PAD // NSUB
LANES = 16
HW = H // 2
NBLK = H // 32

_HIMASK = -65536


def _pack_words(v):
    lo = jnp.concatenate([v[:, 32 * b:32 * b + 16] for b in range(NBLK)],
                         axis=1)
    hi = jnp.concatenate([v[:, 32 * b + 16:32 * b + 32] for b in range(NBLK)],
                         axis=1)
    bl = lax.bitcast_convert_type(lo, jnp.int32) + 0x8000
    bh = lax.bitcast_convert_type(hi, jnp.int32) + 0x8000
    return (bh & _HIMASK) | lax.shift_right_logical(bl, 16)


def _emb_kernel(z_ref, w0_ref, b0_ref, w1_ref, b1_ref, o_ref, ow_ref):
    h = jnp.maximum(
        jnp.dot(z_ref[...], w0_ref[...], preferred_element_type=jnp.float32)
        + b0_ref[...], 0.0)
    y = jnp.dot(h, w1_ref[...],
                preferred_element_type=jnp.float32) + b1_ref[...]
    o_ref[...] = y
    ow_ref[...] = _pack_words(y)


def _edge_kernel(ea_ref, el_ref, w0_ref, b0_ref, w1_ref, b1_ref, o_ref):
    h = jnp.maximum(
        jnp.dot(ea_ref[...], w0_ref[...], preferred_element_type=jnp.float32)
        + b0_ref[...], 0.0)
    y = jnp.dot(h, w1_ref[...],
                preferred_element_type=jnp.float32) + b1_ref[...]
    o_ref[...] = _pack_words(jnp.where(el_ref[...] <= CUTOFF, y, 0.0))


def _update_kernel(parts_ref0, parts_ref1, cv_ref, w0_ref, b0_ref,
                   w1_ref, b1_ref, o_ref, ow_ref, *, apply_relu):
    cv = cv_ref[...]
    out = parts_ref0[0] + parts_ref1[0] + cv
    h = jnp.maximum(
        jnp.dot(out, w0_ref[...], preferred_element_type=jnp.float32)
        + b0_ref[...], 0.0)
    y = jnp.dot(h, w1_ref[...],
                preferred_element_type=jnp.float32) + b1_ref[...]
    if apply_relu:
        y = jnp.maximum(y, 0.0)
    y = y + cv
    o_ref[...] = y
    ow_ref[...] = _pack_words(y)


def _full_spec(shape):
    return pl.BlockSpec(shape, lambda i: (0,) * len(shape))


def _emb(z, w0, b0, w1, b1):
    bn = 2000
    return pl.pallas_call(
        _emb_kernel,
        grid=(N // bn,),
        in_specs=[
            pl.BlockSpec((bn, z.shape[1]), lambda i: (i, 0)),
            _full_spec(w0.shape), _full_spec(b0.shape),
            _full_spec(w1.shape), _full_spec(b1.shape),
        ],
        out_specs=[pl.BlockSpec((bn, H), lambda i: (i, 0)),
                   pl.BlockSpec((bn, HW), lambda i: (i, 0))],
        out_shape=[jax.ShapeDtypeStruct((N, H), jnp.float32),
                   jax.ShapeDtypeStruct((N, HW), jnp.int32)],
    )(z, w0, b0, w1, b1)


def _edge_mlp(ea, el, w0, b0, w1, b1):
    be = 4000
    return pl.pallas_call(
        _edge_kernel,
        grid=(E // be,),
        in_specs=[
            pl.BlockSpec((be, H), lambda i: (i, 0)),
            pl.BlockSpec((be, 1), lambda i: (i, 0)),
            _full_spec(w0.shape), _full_spec(b0.shape),
            _full_spec(w1.shape), _full_spec(b1.shape),
        ],
        out_specs=pl.BlockSpec((be, HW), lambda i: (i, 0)),
        out_shape=jax.ShapeDtypeStruct((E, HW), jnp.int32),
    )(ea, el, w0, b0, w1, b1)


def _update(parts, cv, w0, b0, w1, b1, apply_relu):
    bn = 2000
    return pl.pallas_call(
        functools.partial(_update_kernel, apply_relu=apply_relu),
        grid=(N // bn,),
        in_specs=[
            pl.BlockSpec((1, bn, H), lambda i: (0, i, 0)),
            pl.BlockSpec((1, bn, H), lambda i: (1, i, 0)),
            pl.BlockSpec((bn, H), lambda i: (i, 0)),
            _full_spec(w0.shape), _full_spec(b0.shape),
            _full_spec(w1.shape), _full_spec(b1.shape),
        ],
        out_specs=[pl.BlockSpec((bn, H), lambda i: (i, 0)),
                   pl.BlockSpec((bn, HW), lambda i: (i, 0))],
        out_shape=[jax.ShapeDtypeStruct((N, H), jnp.float32),
                   jax.ShapeDtypeStruct((N, HW), jnp.int32)],
    )(parts, parts, cv, w0, b0, w1, b1)


@functools.cache
def _make_sc_propagate():
    mesh = plsc.VectorSubcoreMesh(core_axis_name="c", subcore_axis_name="s",
                                  num_cores=NCORE, num_subcores=NSUB)
    return pl.kernel(
        _sc_propagate_body,
        out_type=jax.ShapeDtypeStruct((NCORE, N_PAD, H), jnp.float32),
        mesh=mesh,
        compiler_params=pltpu.CompilerParams(use_tc_tiling_on_sc=False,
                                             needs_layout_passes=False),
        scratch_types=[
            pltpu.VMEM((1, K), jnp.int32),
            pltpu.VMEM((1, K), jnp.int32),
            pltpu.VMEM((1, K), jnp.int32),
            pltpu.VMEM((1, K), jnp.int32),
            pltpu.VMEM((1, K), jnp.int32),
            pltpu.VMEM((1, K), jnp.int32),
            pltpu.VMEM((K, HW), jnp.int32),
            pltpu.VMEM((K, HW), jnp.int32),
            pltpu.VMEM((K, HW), jnp.int32),
            pltpu.VMEM((K, HW), jnp.int32),
            pltpu.VMEM((K, H), jnp.float32),
            pltpu.VMEM((K, H), jnp.float32),
            pltpu.VMEM_SHARED((N_PAD, H), jnp.float32),
            pltpu.SemaphoreType.DMA,
            pltpu.SemaphoreType.DMA,
            pltpu.SemaphoreType.DMA,
            pltpu.SemaphoreType.DMA,
            pltpu.SemaphoreType.DMA,
            pltpu.SemaphoreType.DMA,
        ],
    )


def _sc_propagate_body(x_hbm, wm_hbm, src_hbm, dst_hbm, out_hbm,
                       sv0, sv1, dv0, dv1, dv2, dv3,
                       xb0, xb1, wb0, wb1, mb0, mb1,
                       acc, si0, si1, sd0, sd1, ss0, ss1):
    cid = lax.axis_index("c")
    sid = lax.axis_index("s")
    wid = cid * NSUB + sid
    SV = (sv0, sv1)
    DV = (dv0, dv1, dv2, dv3)
    XB, WB, MB = (xb0, xb1), (wb0, wb1), (mb0, mb1)
    SI, SD, SS = (si0, si1), (sd0, sd1), (ss0, ss1)

    @plsc.parallel_loop(0, K)
    def _zero_row(e):
        for kk in range(H // LANES):
            mb0[e, pl.ds(kk * LANES, LANES)] = jnp.zeros((LANES,), jnp.float32)
    for r in range(ROWS_PER_TILE // K):
        pltpu.sync_copy(mb0, acc.at[pl.ds(sid * ROWS_PER_TILE + r * K, K)])

    plsc.subcore_barrier()

    def _issue_idx(j, b2, b4):
        pltpu.async_copy(src_hbm.at[wid, j], SV[b2], SI[b2])
        pltpu.async_copy(dst_hbm.at[wid, j], DV[b4], SI[b2])

    def _wait_idx(b2, b4):
        pltpu.make_async_copy(src_hbm.at[wid, 0], SV[b2], SI[b2]).wait()
        pltpu.make_async_copy(dst_hbm.at[wid, 0], DV[b4], SI[b2]).wait()

    def _issue_data(j, b2):
        pltpu.async_copy(x_hbm.at[SV[b2].at[0]], XB[b2], SD[b2])
        pltpu.async_copy(wm_hbm.at[wid, j], WB[b2], SD[b2])

    def _wait_data(b2):
        pltpu.make_async_copy(wm_hbm.at[wid, 0], XB[b2], SD[b2]).wait()
        pltpu.make_async_copy(wm_hbm.at[wid, 0], WB[b2], SD[b2]).wait()

    def _compute(b2):
        xbuf, wbuf, mbuf = XB[b2], WB[b2], MB[b2]

        @plsc.parallel_loop(0, K, unroll=4)
        def _row(e):
            for kk in range(NBLK):
                sl = pl.ds(kk * LANES, LANES)
                xw = xbuf[e, sl]
                ww = wbuf[e, sl]
                x_lo = plsc.bitcast(xw << 16, jnp.float32)
                x_hi = plsc.bitcast(xw & _HIMASK, jnp.float32)
                w_lo = plsc.bitcast(ww << 16, jnp.float32)
                w_hi = plsc.bitcast(ww & _HIMASK, jnp.float32)
                mbuf[e, pl.ds(kk * 32, LANES)] = \
                    jnp.maximum(x_lo + w_lo, 0.0)
                mbuf[e, pl.ds(kk * 32 + LANES, LANES)] = \
                    jnp.maximum(x_hi + w_hi, 0.0)

    def _issue_scatter(b2, b4):
        pltpu.async_copy(MB[b2], acc.at[DV[b4].at[0]], SS[b2], add=True)

    def _wait_scatter(b2):
        pltpu.make_async_copy(MB[b2], acc.at[pl.ds(0, K)], SS[b2]).wait()

    _issue_idx(0, 0, 0)
    _wait_idx(0, 0)
    _issue_data(0, 0)
    _issue_idx(1, 1, 1)

    def _steady(j2, _):
        for b in range(4):
            j = j2 * 4 + b
            b2, bn2, b4 = b % 2, (b + 1) % 2, b
            _wait_data(b2)
            _wait_idx(bn2, (b4 + 1) % 4)
            if b == 0:
                @pl.when(j2 > 0)
                def _():
                    _wait_scatter(bn2)
            else:
                _wait_scatter(bn2)
            _issue_data(j + 1, bn2)
            _compute(b2)
            _issue_scatter(b2, b4)
            if b == 3:
                @pl.when(j2 < (NCH - 1) // 4 - 1)
                def _():
                    _issue_idx(j + 2, b2, (b4 + 2) % 4)
            else:
                _issue_idx(j + 2, b2, (b4 + 2) % 4)
        return 0

    lax.fori_loop(0, (NCH - 1) // 4, _steady, 0)
    _wait_data(0)
    _wait_scatter(1)
    _compute(0)
    _issue_scatter(0, 0)
    _wait_scatter(0)
    plsc.subcore_barrier()

    sl = pl.ds(sid * ROWS_PER_TILE, ROWS_PER_TILE)
    pltpu.sync_copy(acc.at[sl], out_hbm.at[cid, sl])


def kernel(z, edge_index, edge_attr, edge_length,
           emb_W0, emb_b0, emb_W1, emb_b1,
           m1_W0, m1_b0, m1_W1, m1_b1,
           m2_W0, m2_b0, m2_W1, m2_b1):
    x, xw = _emb(z, emb_W0, emb_b0.reshape(1, H), emb_W1, emb_b1.reshape(1, H))
    wm_w = _edge_mlp(edge_attr, edge_length.reshape(E, 1),
                     m2_W0, m2_b0.reshape(1, H), m2_W1, m2_b1.reshape(1, H)
                     ).reshape(NW, NCH, K, HW)
    src = edge_index[0].astype(jnp.int32).reshape(NW, NCH, 1, K)
    dst = edge_index[1].astype(jnp.int32).reshape(NW, NCH, 1, K)

    conv = x
    for conv_id in range(NUM_CONVS):
        parts = _make_sc_propagate()(xw, wm_w, src, dst)
        conv, xw = _update(parts, conv,
                           m1_W0, m1_b0.reshape(1, H),
                           m1_W1, m1_b1.reshape(1, H),
                           apply_relu=conv_id < NUM_CONVS - 1)
    return conv


# speedup vs baseline: 1.8571x; 1.0662x over previous
import functools

import jax
import jax.numpy as jnp
from jax import lax
from jax.experimental import pallas as pl
from jax.experimental.pallas import tpu as pltpu
from jax.experimental.pallas import tpu_sc as plsc

N = 10000
E = 320000
H = 128
CUTOFF = 10.0
NUM_CONVS = 3

NCORE = 2
NSUB = 16
NW = NCORE * NSUB
EW = E // NW
K = 80
NCH = EW // K
N_PAD = 10240
ROWS_PER_TILE = N_PAD // NSUB
LANES = 16
HW = H // 2
NBLK = H // 32

_HIMASK = -65536


def _pack_words(v):
    bl = lax.bitcast_convert_type(v[:, :HW], jnp.int32) + 0x8000
    bh = lax.bitcast_convert_type(v[:, HW:], jnp.int32) + 0x8000
    return (bh & _HIMASK) | lax.shift_right_logical(bl, 16)


def _emb_kernel(z_ref, w0_ref, b0_ref, w1_ref, b1_ref, o_ref, ow_ref):
    h = jnp.maximum(
        jnp.dot(z_ref[...], w0_ref[...], preferred_element_type=jnp.float32)
        + b0_ref[...], 0.0)
    y = jnp.dot(h, w1_ref[...],
                preferred_element_type=jnp.float32) + b1_ref[...]
    o_ref[...] = y
    ow_ref[...] = _pack_words(y)


def _edge_kernel(ea_ref, el_ref, w0_ref, b0_ref, w1_ref, b1_ref, o_ref):
    h = jnp.maximum(
        jnp.dot(ea_ref[...], w0_ref[...], preferred_element_type=jnp.float32)
        + b0_ref[...], 0.0)
    y = jnp.dot(h, w1_ref[...],
                preferred_element_type=jnp.float32) + b1_ref[...]
    o_ref[...] = _pack_words(jnp.where(el_ref[...] <= CUTOFF, y, 0.0))


def _update_kernel(parts_ref0, parts_ref1, cv_ref, w0_ref, b0_ref,
                   w1_ref, b1_ref, o_ref, ow_ref, *, apply_relu):
    cv = cv_ref[...]
    out = parts_ref0[0] + parts_ref1[0] + cv
    h = jnp.maximum(
        jnp.dot(out, w0_ref[...], preferred_element_type=jnp.float32)
        + b0_ref[...], 0.0)
    y = jnp.dot(h, w1_ref[...],
                preferred_element_type=jnp.float32) + b1_ref[...]
    if apply_relu:
        y = jnp.maximum(y, 0.0)
    y = y + cv
    o_ref[...] = y
    ow_ref[...] = _pack_words(y)


def _full_spec(shape):
    return pl.BlockSpec(shape, lambda i: (0,) * len(shape))


def _emb(z, w0, b0, w1, b1):
    bn = 2000
    return pl.pallas_call(
        _emb_kernel,
        grid=(N // bn,),
        in_specs=[
            pl.BlockSpec((bn, z.shape[1]), lambda i: (i, 0)),
            _full_spec(w0.shape), _full_spec(b0.shape),
            _full_spec(w1.shape), _full_spec(b1.shape),
        ],
        out_specs=[pl.BlockSpec((bn, H), lambda i: (i, 0)),
                   pl.BlockSpec((bn, HW), lambda i: (i, 0))],
        out_shape=[jax.ShapeDtypeStruct((N, H), jnp.float32),
                   jax.ShapeDtypeStruct((N, HW), jnp.int32)],
    )(z, w0, b0, w1, b1)


def _edge_mlp(ea, el, w0, b0, w1, b1):
    be = 4000
    return pl.pallas_call(
        _edge_kernel,
        grid=(E // be,),
        in_specs=[
            pl.BlockSpec((be, H), lambda i: (i, 0)),
            pl.BlockSpec((be, 1), lambda i: (i, 0)),
            _full_spec(w0.shape), _full_spec(b0.shape),
            _full_spec(w1.shape), _full_spec(b1.shape),
        ],
        out_specs=pl.BlockSpec((be, HW), lambda i: (i, 0)),
        out_shape=jax.ShapeDtypeStruct((E, HW), jnp.int32),
    )(ea, el, w0, b0, w1, b1)


def _update(parts, cv, w0, b0, w1, b1, apply_relu):
    bn = 2000
    return pl.pallas_call(
        functools.partial(_update_kernel, apply_relu=apply_relu),
        grid=(N // bn,),
        in_specs=[
            pl.BlockSpec((1, bn, H), lambda i: (0, i, 0)),
            pl.BlockSpec((1, bn, H), lambda i: (1, i, 0)),
            pl.BlockSpec((bn, H), lambda i: (i, 0)),
            _full_spec(w0.shape), _full_spec(b0.shape),
            _full_spec(w1.shape), _full_spec(b1.shape),
        ],
        out_specs=[pl.BlockSpec((bn, H), lambda i: (i, 0)),
                   pl.BlockSpec((bn, HW), lambda i: (i, 0))],
        out_shape=[jax.ShapeDtypeStruct((N, H), jnp.float32),
                   jax.ShapeDtypeStruct((N, HW), jnp.int32)],
    )(parts, parts, cv, w0, b0, w1, b1)


@functools.cache
def _make_sc_propagate():
    mesh = plsc.VectorSubcoreMesh(core_axis_name="c", subcore_axis_name="s",
                                  num_cores=NCORE, num_subcores=NSUB)
    return pl.kernel(
        _sc_propagate_body,
        out_type=jax.ShapeDtypeStruct((NCORE, N_PAD, H), jnp.float32),
        mesh=mesh,
        compiler_params=pltpu.CompilerParams(use_tc_tiling_on_sc=False,
                                             needs_layout_passes=False),
        scratch_types=[
            pltpu.VMEM((1, K), jnp.int32),
            pltpu.VMEM((1, K), jnp.int32),
            pltpu.VMEM((1, K), jnp.int32),
            pltpu.VMEM((1, K), jnp.int32),
            pltpu.VMEM((1, K), jnp.int32),
            pltpu.VMEM((1, K), jnp.int32),
            pltpu.VMEM((K, HW), jnp.int32),
            pltpu.VMEM((K, HW), jnp.int32),
            pltpu.VMEM((K, HW), jnp.int32),
            pltpu.VMEM((K, HW), jnp.int32),
            pltpu.VMEM((K, H), jnp.float32),
            pltpu.VMEM((K, H), jnp.float32),
            pltpu.VMEM_SHARED((N_PAD, H), jnp.float32),
            pltpu.SemaphoreType.DMA,
            pltpu.SemaphoreType.DMA,
            pltpu.SemaphoreType.DMA,
            pltpu.SemaphoreType.DMA,
            pltpu.SemaphoreType.DMA,
            pltpu.SemaphoreType.DMA,
        ],
    )


def _sc_propagate_body(x_hbm, wm_hbm, src_hbm, dst_hbm, out_hbm,
                       sv0, sv1, dv0, dv1, dv2, dv3,
                       xb0, xb1, wb0, wb1, mb0, mb1,
                       acc, si0, si1, sd0, sd1, ss0, ss1):
    cid = lax.axis_index("c")
    sid = lax.axis_index("s")
    wid = cid * NSUB + sid
    SV = (sv0, sv1)
    DV = (dv0, dv1, dv2, dv3)
    XB, WB, MB = (xb0, xb1), (wb0, wb1), (mb0, mb1)
    SI, SD, SS = (si0, si1), (sd0, sd1), (ss0, ss1)

    @plsc.parallel_loop(0, K)
    def _zero_row(e):
        for kk in range(H // LANES):
            mb0[e, pl.ds(kk * LANES, LANES)] = jnp.zeros((LANES,), jnp.float32)
    for r in range(ROWS_PER_TILE // K):
        pltpu.sync_copy(mb0, acc.at[pl.ds(sid * ROWS_PER_TILE + r * K, K)])

    plsc.subcore_barrier()

    def _issue_idx(j, b2, b4):
        pltpu.async_copy(src_hbm.at[wid, j], SV[b2], SI[b2])
        pltpu.async_copy(dst_hbm.at[wid, j], DV[b4], SI[b2])

    def _wait_idx(b2, b4):
        pltpu.make_async_copy(src_hbm.at[wid, 0], SV[b2], SI[b2]).wait()
        pltpu.make_async_copy(dst_hbm.at[wid, 0], DV[b4], SI[b2]).wait()

    def _issue_data(j, b2):
        pltpu.async_copy(x_hbm.at[SV[b2].at[0]], XB[b2], SD[b2])
        pltpu.async_copy(wm_hbm.at[wid, j], WB[b2], SD[b2])

    def _wait_data(b2):
        pltpu.make_async_copy(wm_hbm.at[wid, 0], XB[b2], SD[b2]).wait()
        pltpu.make_async_copy(wm_hbm.at[wid, 0], WB[b2], SD[b2]).wait()

    def _compute(b2):
        xbuf, wbuf, mbuf = XB[b2], WB[b2], MB[b2]

        @plsc.parallel_loop(0, K, unroll=4)
        def _row(e):
            for kk in range(HW // LANES):
                sl = pl.ds(kk * LANES, LANES)
                xw = xbuf[e, sl]
                ww = wbuf[e, sl]
                x_lo = plsc.bitcast(xw << 16, jnp.float32)
                x_hi = plsc.bitcast(xw & _HIMASK, jnp.float32)
                w_lo = plsc.bitcast(ww << 16, jnp.float32)
                w_hi = plsc.bitcast(ww & _HIMASK, jnp.float32)
                mbuf[e, sl] = jnp.maximum(x_lo + w_lo, 0.0)
                mbuf[e, pl.ds(HW + kk * LANES, LANES)] = \
                    jnp.maximum(x_hi + w_hi, 0.0)

    def _issue_scatter(b2, b4):
        pltpu.async_copy(MB[b2], acc.at[DV[b4].at[0]], SS[b2], add=True)

    def _wait_scatter(b2):
        pltpu.make_async_copy(MB[b2], acc.at[pl.ds(0, K)], SS[b2]).wait()

    _issue_idx(0, 0, 0)
    _wait_idx(0, 0)
    _issue_data(0, 0)
    _issue_idx(1, 1, 1)

    def _steady(j2, _):
        for b in range(4):
            j = j2 * 4 + b
            b2, bn2, b4 = b % 2, (b + 1) % 2, b
            _wait_data(b2)
            _wait_idx(bn2, (b4 + 1) % 4)
            if b == 0:
                @pl.when(j2 > 0)
                def _():
                    _wait_scatter(bn2)
            else:
                _wait_scatter(bn2)
            _issue_data(j + 1, bn2)
            _compute(b2)
            _issue_scatter(b2, b4)
            if b == 3:
                @pl.when(j2 < (NCH - 1) // 4 - 1)
                def _():
                    _issue_idx(j + 2, b2, (b4 + 2) % 4)
            else:
                _issue_idx(j + 2, b2, (b4 + 2) % 4)
        return 0

    lax.fori_loop(0, (NCH - 1) // 4, _steady, 0)
    _wait_data(0)
    _wait_scatter(1)
    _compute(0)
    _issue_scatter(0, 0)
    _wait_scatter(0)
    plsc.subcore_barrier()

    sl = pl.ds(sid * ROWS_PER_TILE, ROWS_PER_TILE)
    pltpu.sync_copy(acc.at[sl], out_hbm.at[cid, sl])


def kernel(z, edge_index, edge_attr, edge_length,
           emb_W0, emb_b0, emb_W1, emb_b1,
           m1_W0, m1_b0, m1_W1, m1_b1,
           m2_W0, m2_b0, m2_W1, m2_b1):
    x, xw = _emb(z, emb_W0, emb_b0.reshape(1, H), emb_W1, emb_b1.reshape(1, H))
    wm_w = _edge_mlp(edge_attr, edge_length.reshape(E, 1),
                     m2_W0, m2_b0.reshape(1, H), m2_W1, m2_b1.reshape(1, H)
                     ).reshape(NW, NCH, K, HW)
    src = edge_index[0].astype(jnp.int32).reshape(NW, NCH, 1, K)
    dst = edge_index[1].astype(jnp.int32).reshape(NW, NCH, 1, K)

    conv = x
    for conv_id in range(NUM_CONVS):
        parts = _make_sc_propagate()(xw, wm_w, src, dst)
        conv, xw = _update(parts, conv,
                           m1_W0, m1_b0.reshape(1, H),
                           m1_W1, m1_b1.reshape(1, H),
                           apply_relu=conv_id < NUM_CONVS - 1)
    return conv


# speedup vs baseline: 1.8992x; 1.0226x over previous
import functools

import jax
import jax.numpy as jnp
from jax import lax
from jax.experimental import pallas as pl
from jax.experimental.pallas import tpu as pltpu
from jax.experimental.pallas import tpu_sc as plsc

N = 10000
E = 320000
H = 128
CUTOFF = 10.0
NUM_CONVS = 3

NCORE = 2
NSUB = 16
NW = NCORE * NSUB
EW = E // NW
K = 80
NCH = EW // K
N_PAD = 10240
ROWS_PER_TILE = N_PAD // NSUB
LANES = 16
HW = H // 2
NBLK = H // 32

_HIMASK = -65536


def _pack_words(v):
    bl = lax.bitcast_convert_type(v[:, :HW], jnp.int32)
    bh = lax.bitcast_convert_type(v[:, HW:], jnp.int32)
    return (bh & _HIMASK) | lax.shift_right_logical(bl, 16)


def _emb_kernel(z_ref, w0_ref, b0_ref, w1_ref, b1_ref, o_ref, ow_ref):
    h = jnp.maximum(
        jnp.dot(z_ref[...], w0_ref[...], preferred_element_type=jnp.float32)
        + b0_ref[...], 0.0)
    y = jnp.dot(h, w1_ref[...],
                preferred_element_type=jnp.float32) + b1_ref[...]
    o_ref[...] = y
    ow_ref[...] = _pack_words(y)


def _edge_kernel(ea_ref, el_ref, w0_ref, b0_ref, w1_ref, b1_ref, o_ref):
    h = jnp.maximum(
        jnp.dot(ea_ref[...].astype(jnp.bfloat16), w0_ref[...],
                preferred_element_type=jnp.float32) + b0_ref[...], 0.0)
    y = jnp.dot(h.astype(jnp.bfloat16), w1_ref[...],
                preferred_element_type=jnp.float32) + b1_ref[...]
    mask = jnp.where(el_ref[...] <= CUTOFF, -1, 0)
    o_ref[...] = _pack_words(y) & mask


def _update_kernel(parts_ref0, parts_ref1, cv_ref, w0_ref, b0_ref,
                   w1_ref, b1_ref, o_ref, ow_ref, *, apply_relu):
    cv = cv_ref[...]
    out = parts_ref0[0] + parts_ref1[0] + cv
    h = jnp.maximum(
        jnp.dot(out, w0_ref[...], preferred_element_type=jnp.float32)
        + b0_ref[...], 0.0)
    y = jnp.dot(h, w1_ref[...],
                preferred_element_type=jnp.float32) + b1_ref[...]
    if apply_relu:
        y = jnp.maximum(y, 0.0)
    y = y + cv
    o_ref[...] = y
    ow_ref[...] = _pack_words(y)


def _full_spec(shape):
    return pl.BlockSpec(shape, lambda i: (0,) * len(shape))


def _emb(z, w0, b0, w1, b1):
    bn = 2000
    return pl.pallas_call(
        _emb_kernel,
        grid=(N // bn,),
        in_specs=[
            pl.BlockSpec((bn, z.shape[1]), lambda i: (i, 0)),
            _full_spec(w0.shape), _full_spec(b0.shape),
            _full_spec(w1.shape), _full_spec(b1.shape),
        ],
        out_specs=[pl.BlockSpec((bn, H), lambda i: (i, 0)),
                   pl.BlockSpec((bn, HW), lambda i: (i, 0))],
        out_shape=[jax.ShapeDtypeStruct((N, H), jnp.float32),
                   jax.ShapeDtypeStruct((N, HW), jnp.int32)],
    )(z, w0, b0, w1, b1)


def _edge_mlp(ea, el, w0, b0, w1, b1):
    be = 8000
    return pl.pallas_call(
        _edge_kernel,
        grid=(E // be,),
        in_specs=[
            pl.BlockSpec((be, H), lambda i: (i, 0)),
            pl.BlockSpec((be, 1), lambda i: (i, 0)),
            _full_spec(w0.shape), _full_spec(b0.shape),
            _full_spec(w1.shape), _full_spec(b1.shape),
        ],
        out_specs=pl.BlockSpec((be, HW), lambda i: (i, 0)),
        out_shape=jax.ShapeDtypeStruct((E, HW), jnp.int32),
    )(ea, el, w0, b0, w1, b1)


def _update(parts, cv, w0, b0, w1, b1, apply_relu):
    bn = 2000
    return pl.pallas_call(
        functools.partial(_update_kernel, apply_relu=apply_relu),
        grid=(N // bn,),
        in_specs=[
            pl.BlockSpec((1, bn, H), lambda i: (0, i, 0)),
            pl.BlockSpec((1, bn, H), lambda i: (1, i, 0)),
            pl.BlockSpec((bn, H), lambda i: (i, 0)),
            _full_spec(w0.shape), _full_spec(b0.shape),
            _full_spec(w1.shape), _full_spec(b1.shape),
        ],
        out_specs=[pl.BlockSpec((bn, H), lambda i: (i, 0)),
                   pl.BlockSpec((bn, HW), lambda i: (i, 0))],
        out_shape=[jax.ShapeDtypeStruct((N, H), jnp.float32),
                   jax.ShapeDtypeStruct((N, HW), jnp.int32)],
    )(parts, parts, cv, w0, b0, w1, b1)


@functools.cache
def _make_sc_propagate():
    mesh = plsc.VectorSubcoreMesh(core_axis_name="c", subcore_axis_name="s",
                                  num_cores=NCORE, num_subcores=NSUB)
    return pl.kernel(
        _sc_propagate_body,
        out_type=jax.ShapeDtypeStruct((NCORE, N_PAD, H), jnp.float32),
        mesh=mesh,
        compiler_params=pltpu.CompilerParams(use_tc_tiling_on_sc=False,
                                             needs_layout_passes=False),
        scratch_types=[
            pltpu.VMEM((1, K), jnp.int32),
            pltpu.VMEM((1, K), jnp.int32),
            pltpu.VMEM((1, K), jnp.int32),
            pltpu.VMEM((1, K), jnp.int32),
            pltpu.VMEM((1, K), jnp.int32),
            pltpu.VMEM((1, K), jnp.int32),
            pltpu.VMEM((K, HW), jnp.int32),
            pltpu.VMEM((K, HW), jnp.int32),
            pltpu.VMEM((K, HW), jnp.int32),
            pltpu.VMEM((K, HW), jnp.int32),
            pltpu.VMEM((K, H), jnp.float32),
            pltpu.VMEM((K, H), jnp.float32),
            pltpu.VMEM_SHARED((N_PAD, H), jnp.float32),
            pltpu.SemaphoreType.DMA,
            pltpu.SemaphoreType.DMA,
            pltpu.SemaphoreType.DMA,
            pltpu.SemaphoreType.DMA,
            pltpu.SemaphoreType.DMA,
            pltpu.SemaphoreType.DMA,
        ],
    )


def _sc_propagate_body(x_hbm, wm_hbm, src_hbm, dst_hbm, out_hbm,
                       sv0, sv1, dv0, dv1, dv2, dv3,
                       xb0, xb1, wb0, wb1, mb0, mb1,
                       acc, si0, si1, sd0, sd1, ss0, ss1):
    cid = lax.axis_index("c")
    sid = lax.axis_index("s")
    wid = cid * NSUB + sid
    SV = (sv0, sv1)
    DV = (dv0, dv1, dv2, dv3)
    XB, WB, MB = (xb0, xb1), (wb0, wb1), (mb0, mb1)
    SI, SD, SS = (si0, si1), (sd0, sd1), (ss0, ss1)

    @plsc.parallel_loop(0, K)
    def _zero_row(e):
        for kk in range(H // LANES):
            mb0[e, pl.ds(kk * LANES, LANES)] = jnp.zeros((LANES,), jnp.float32)
    for r in range(ROWS_PER_TILE // K):
        pltpu.sync_copy(mb0, acc.at[pl.ds(sid * ROWS_PER_TILE + r * K, K)])

    plsc.subcore_barrier()

    def _issue_idx(j, b2, b4):
        pltpu.async_copy(src_hbm.at[wid, j], SV[b2], SI[b2])
        pltpu.async_copy(dst_hbm.at[wid, j], DV[b4], SI[b2])

    def _wait_idx(b2, b4):
        pltpu.make_async_copy(src_hbm.at[wid, 0], SV[b2], SI[b2]).wait()
        pltpu.make_async_copy(dst_hbm.at[wid, 0], DV[b4], SI[b2]).wait()

    def _issue_data(j, b2):
        pltpu.async_copy(x_hbm.at[SV[b2].at[0]], XB[b2], SD[b2])
        pltpu.async_copy(wm_hbm.at[wid, j], WB[b2], SD[b2])

    def _wait_data(b2):
        pltpu.make_async_copy(wm_hbm.at[wid, 0], XB[b2], SD[b2]).wait()
        pltpu.make_async_copy(wm_hbm.at[wid, 0], WB[b2], SD[b2]).wait()

    def _compute(b2):
        xbuf, wbuf, mbuf = XB[b2], WB[b2], MB[b2]

        @plsc.parallel_loop(0, K, unroll=4)
        def _row(e):
            for kk in range(HW // LANES):
                sl = pl.ds(kk * LANES, LANES)
                xw = xbuf[e, sl]
                ww = wbuf[e, sl]
                x_lo = plsc.bitcast(xw << 16, jnp.float32)
                x_hi = plsc.bitcast(xw & _HIMASK, jnp.float32)
                w_lo = plsc.bitcast(ww << 16, jnp.float32)
                w_hi = plsc.bitcast(ww & _HIMASK, jnp.float32)
                mbuf[e, sl] = jnp.maximum(x_lo + w_lo, 0.0)
                mbuf[e, pl.ds(HW + kk * LANES, LANES)] = \
                    jnp.maximum(x_hi + w_hi, 0.0)

    def _issue_scatter(b2, b4):
        pltpu.async_copy(MB[b2], acc.at[DV[b4].at[0]], SS[b2], add=True)

    def _wait_scatter(b2):
        pltpu.make_async_copy(MB[b2], acc.at[pl.ds(0, K)], SS[b2]).wait()

    _issue_idx(0, 0, 0)
    _wait_idx(0, 0)
    _issue_data(0, 0)
    _issue_idx(1, 1, 1)

    def _steady(j2, _):
        for b in range(4):
            j = j2 * 4 + b
            b2, bn2, b4 = b % 2, (b + 1) % 2, b
            _wait_data(b2)
            _wait_idx(bn2, (b4 + 1) % 4)
            if b == 0:
                @pl.when(j2 > 0)
                def _():
                    _wait_scatter(bn2)
            else:
                _wait_scatter(bn2)
            _issue_data(j + 1, bn2)
            _compute(b2)
            _issue_scatter(b2, b4)
            if b == 3:
                @pl.when(j2 < (NCH - 1) // 4 - 1)
                def _():
                    _issue_idx(j + 2, b2, (b4 + 2) % 4)
            else:
                _issue_idx(j + 2, b2, (b4 + 2) % 4)
        return 0

    lax.fori_loop(0, (NCH - 1) // 4, _steady, 0)
    _wait_data(0)
    _wait_scatter(1)
    _compute(0)
    _issue_scatter(0, 0)
    _wait_scatter(0)
    plsc.subcore_barrier()

    sl = pl.ds(sid * ROWS_PER_TILE, ROWS_PER_TILE)
    pltpu.sync_copy(acc.at[sl], out_hbm.at[cid, sl])


def kernel(z, edge_index, edge_attr, edge_length,
           emb_W0, emb_b0, emb_W1, emb_b1,
           m1_W0, m1_b0, m1_W1, m1_b1,
           m2_W0, m2_b0, m2_W1, m2_b1):
    x, xw = _emb(z, emb_W0, emb_b0.reshape(1, H), emb_W1, emb_b1.reshape(1, H))
    wm_w = _edge_mlp(edge_attr, edge_length.reshape(E, 1),
                     m2_W0.astype(jnp.bfloat16), m2_b0.reshape(1, H),
                     m2_W1.astype(jnp.bfloat16), m2_b1.reshape(1, H)
                     ).reshape(NW, NCH, K, HW)
    src = edge_index[0].astype(jnp.int32).reshape(NW, NCH, 1, K)
    dst = edge_index[1].astype(jnp.int32).reshape(NW, NCH, 1, K)

    conv = x
    for conv_id in range(NUM_CONVS):
        parts = _make_sc_propagate()(xw, wm_w, src, dst)
        conv, xw = _update(parts, conv,
                           m1_W0, m1_b0.reshape(1, H),
                           m1_W1, m1_b1.reshape(1, H),
                           apply_relu=conv_id < NUM_CONVS - 1)
    return conv


# speedup vs baseline: 1.9056x; 1.0034x over previous
import functools

import jax
import jax.numpy as jnp
from jax import lax
from jax.experimental import pallas as pl
from jax.experimental.pallas import tpu as pltpu
from jax.experimental.pallas import tpu_sc as plsc

N = 10000
E = 320000
H = 128
CUTOFF = 10.0
NUM_CONVS = 3

NCORE = 2
NSUB = 16
NW = NCORE * NSUB
EW = E // NW
K = 80
NCH = EW // K
N_PAD = 10240
ROWS_PER_TILE = N_PAD // NSUB
LANES = 16
HW = H // 2
NBLK = H // 32

_HIMASK = -65536


def _pack_words(v):
    bl = lax.bitcast_convert_type(v[:, :HW], jnp.int32) + 0x8000
    bh = lax.bitcast_convert_type(v[:, HW:], jnp.int32) + 0x8000
    return (bh & _HIMASK) | lax.shift_right_logical(bl, 16)


def _emb_kernel(z_ref, w0_ref, b0_ref, w1_ref, b1_ref, o_ref, ow_ref):
    h = jnp.maximum(
        jnp.dot(z_ref[...], w0_ref[...], preferred_element_type=jnp.float32)
        + b0_ref[...], 0.0)
    y = jnp.dot(h, w1_ref[...],
                preferred_element_type=jnp.float32) + b1_ref[...]
    o_ref[...] = y
    ow_ref[...] = _pack_words(y)


def _edge_kernel(ea_ref, el_ref, w0_ref, b0_ref, w1_ref, b1_ref, o_ref):
    h = jnp.maximum(
        jnp.dot(ea_ref[...], w0_ref[...],
                preferred_element_type=jnp.float32) + b0_ref[...], 0.0)
    y = jnp.dot(h, w1_ref[...],
                preferred_element_type=jnp.float32) + b1_ref[...]
    mask = jnp.where(el_ref[...] <= CUTOFF, -1, 0)
    o_ref[...] = _pack_words(y) & mask


def _update_kernel(parts_ref0, parts_ref1, cv_ref, w0_ref, b0_ref,
                   w1_ref, b1_ref, o_ref, ow_ref, *, apply_relu):
    cv = cv_ref[...]
    out = parts_ref0[0] + parts_ref1[0] + cv
    h = jnp.maximum(
        jnp.dot(out, w0_ref[...], preferred_element_type=jnp.float32)
        + b0_ref[...], 0.0)
    y = jnp.dot(h, w1_ref[...],
                preferred_element_type=jnp.float32) + b1_ref[...]
    if apply_relu:
        y = jnp.maximum(y, 0.0)
    y = y + cv
    o_ref[...] = y
    ow_ref[...] = _pack_words(y)


def _full_spec(shape):
    return pl.BlockSpec(shape, lambda i: (0,) * len(shape))


def _emb(z, w0, b0, w1, b1):
    bn = 2000
    return pl.pallas_call(
        _emb_kernel,
        grid=(N // bn,),
        in_specs=[
            pl.BlockSpec((bn, z.shape[1]), lambda i: (i, 0)),
            _full_spec(w0.shape), _full_spec(b0.shape),
            _full_spec(w1.shape), _full_spec(b1.shape),
        ],
        out_specs=[pl.BlockSpec((bn, H), lambda i: (i, 0)),
                   pl.BlockSpec((bn, HW), lambda i: (i, 0))],
        out_shape=[jax.ShapeDtypeStruct((N, H), jnp.float32),
                   jax.ShapeDtypeStruct((N, HW), jnp.int32)],
    )(z, w0, b0, w1, b1)


def _edge_mlp(ea, el, w0, b0, w1, b1):
    be = 8000
    return pl.pallas_call(
        _edge_kernel,
        grid=(E // be,),
        in_specs=[
            pl.BlockSpec((be, H), lambda i: (i, 0)),
            pl.BlockSpec((be, 1), lambda i: (i, 0)),
            _full_spec(w0.shape), _full_spec(b0.shape),
            _full_spec(w1.shape), _full_spec(b1.shape),
        ],
        out_specs=pl.BlockSpec((be, HW), lambda i: (i, 0)),
        out_shape=jax.ShapeDtypeStruct((E, HW), jnp.int32),
    )(ea, el, w0, b0, w1, b1)


def _update(parts, cv, w0, b0, w1, b1, apply_relu):
    bn = 2000
    return pl.pallas_call(
        functools.partial(_update_kernel, apply_relu=apply_relu),
        grid=(N // bn,),
        in_specs=[
            pl.BlockSpec((1, bn, H), lambda i: (0, i, 0)),
            pl.BlockSpec((1, bn, H), lambda i: (1, i, 0)),
            pl.BlockSpec((bn, H), lambda i: (i, 0)),
            _full_spec(w0.shape), _full_spec(b0.shape),
            _full_spec(w1.shape), _full_spec(b1.shape),
        ],
        out_specs=[pl.BlockSpec((bn, H), lambda i: (i, 0)),
                   pl.BlockSpec((bn, HW), lambda i: (i, 0))],
        out_shape=[jax.ShapeDtypeStruct((N, H), jnp.float32),
                   jax.ShapeDtypeStruct((N, HW), jnp.int32)],
    )(parts, parts, cv, w0, b0, w1, b1)


@functools.cache
def _make_sc_propagate():
    mesh = plsc.VectorSubcoreMesh(core_axis_name="c", subcore_axis_name="s",
                                  num_cores=NCORE, num_subcores=NSUB)
    return pl.kernel(
        _sc_propagate_body,
        out_type=jax.ShapeDtypeStruct((NCORE, N_PAD, H), jnp.float32),
        mesh=mesh,
        compiler_params=pltpu.CompilerParams(use_tc_tiling_on_sc=False,
                                             needs_layout_passes=False),
        scratch_types=[
            pltpu.VMEM((1, K), jnp.int32),
            pltpu.VMEM((1, K), jnp.int32),
            pltpu.VMEM((1, K), jnp.int32),
            pltpu.VMEM((1, K), jnp.int32),
            pltpu.VMEM((1, K), jnp.int32),
            pltpu.VMEM((1, K), jnp.int32),
            pltpu.VMEM((K, HW), jnp.int32),
            pltpu.VMEM((K, HW), jnp.int32),
            pltpu.VMEM((K, HW), jnp.int32),
            pltpu.VMEM((K, HW), jnp.int32),
            pltpu.VMEM((K, H), jnp.float32),
            pltpu.VMEM((K, H), jnp.float32),
            pltpu.VMEM_SHARED((N_PAD, H), jnp.float32),
            pltpu.SemaphoreType.DMA,
            pltpu.SemaphoreType.DMA,
            pltpu.SemaphoreType.DMA,
            pltpu.SemaphoreType.DMA,
            pltpu.SemaphoreType.DMA,
            pltpu.SemaphoreType.DMA,
        ],
    )


def _sc_propagate_body(x_hbm, wm_hbm, src_hbm, dst_hbm, out_hbm,
                       sv0, sv1, dv0, dv1, dv2, dv3,
                       xb0, xb1, wb0, wb1, mb0, mb1,
                       acc, si0, si1, sd0, sd1, ss0, ss1):
    cid = lax.axis_index("c")
    sid = lax.axis_index("s")
    wid = cid * NSUB + sid
    SV = (sv0, sv1)
    DV = (dv0, dv1, dv2, dv3)
    XB, WB, MB = (xb0, xb1), (wb0, wb1), (mb0, mb1)
    SI, SD, SS = (si0, si1), (sd0, sd1), (ss0, ss1)

    @plsc.parallel_loop(0, K)
    def _zero_row(e):
        for kk in range(H // LANES):
            mb0[e, pl.ds(kk * LANES, LANES)] = jnp.zeros((LANES,), jnp.float32)
    for r in range(ROWS_PER_TILE // K):
        pltpu.sync_copy(mb0, acc.at[pl.ds(sid * ROWS_PER_TILE + r * K, K)])

    plsc.subcore_barrier()

    def _issue_idx(j, b2, b4):
        pltpu.async_copy(src_hbm.at[wid, j], SV[b2], SI[b2])
        pltpu.async_copy(dst_hbm.at[wid, j], DV[b4], SI[b2])

    def _wait_idx(b2, b4):
        pltpu.make_async_copy(src_hbm.at[wid, 0], SV[b2], SI[b2]).wait()
        pltpu.make_async_copy(dst_hbm.at[wid, 0], DV[b4], SI[b2]).wait()

    def _issue_data(j, b2):
        pltpu.async_copy(x_hbm.at[SV[b2].at[0]], XB[b2], SD[b2])
        pltpu.async_copy(wm_hbm.at[wid, j], WB[b2], SD[b2])

    def _wait_data(b2):
        pltpu.make_async_copy(wm_hbm.at[wid, 0], XB[b2], SD[b2]).wait()
        pltpu.make_async_copy(wm_hbm.at[wid, 0], WB[b2], SD[b2]).wait()

    def _compute(b2):
        xbuf, wbuf, mbuf = XB[b2], WB[b2], MB[b2]

        @plsc.parallel_loop(0, K, unroll=4)
        def _row(e):
            for kk in range(HW // LANES):
                sl = pl.ds(kk * LANES, LANES)
                xw = xbuf[e, sl]
                ww = wbuf[e, sl]
                x_lo = plsc.bitcast(xw << 16, jnp.float32)
                x_hi = plsc.bitcast(xw & _HIMASK, jnp.float32)
                w_lo = plsc.bitcast(ww << 16, jnp.float32)
                w_hi = plsc.bitcast(ww & _HIMASK, jnp.float32)
                mbuf[e, sl] = jnp.maximum(x_lo + w_lo, 0.0)
                mbuf[e, pl.ds(HW + kk * LANES, LANES)] = \
                    jnp.maximum(x_hi + w_hi, 0.0)

    def _issue_scatter(b2, b4):
        pltpu.async_copy(MB[b2], acc.at[DV[b4].at[0]], SS[b2], add=True)

    def _wait_scatter(b2):
        pltpu.make_async_copy(MB[b2], acc.at[pl.ds(0, K)], SS[b2]).wait()

    _issue_idx(0, 0, 0)
    _wait_idx(0, 0)
    _issue_data(0, 0)
    _issue_idx(1, 1, 1)

    def _steady(j2, _):
        for b in range(4):
            j = j2 * 4 + b
            b2, bn2, b4 = b % 2, (b + 1) % 2, b
            _wait_data(b2)
            _wait_idx(bn2, (b4 + 1) % 4)
            if b == 0:
                @pl.when(j2 > 0)
                def _():
                    _wait_scatter(bn2)
            else:
                _wait_scatter(bn2)
            _issue_data(j + 1, bn2)
            _compute(b2)
            _issue_scatter(b2, b4)
            if b == 3:
                @pl.when(j2 < (NCH - 1) // 4 - 1)
                def _():
                    _issue_idx(j + 2, b2, (b4 + 2) % 4)
            else:
                _issue_idx(j + 2, b2, (b4 + 2) % 4)
        return 0

    lax.fori_loop(0, (NCH - 1) // 4, _steady, 0)
    _wait_data(0)
    _wait_scatter(1)
    _compute(0)
    _issue_scatter(0, 0)
    _wait_scatter(0)
    plsc.subcore_barrier()

    sl = pl.ds(sid * ROWS_PER_TILE, ROWS_PER_TILE)
    pltpu.sync_copy(acc.at[sl], out_hbm.at[cid, sl])


def kernel(z, edge_index, edge_attr, edge_length,
           emb_W0, emb_b0, emb_W1, emb_b1,
           m1_W0, m1_b0, m1_W1, m1_b1,
           m2_W0, m2_b0, m2_W1, m2_b1):
    x, xw = _emb(z, emb_W0, emb_b0.reshape(1, H), emb_W1, emb_b1.reshape(1, H))
    wm_w = _edge_mlp(edge_attr, edge_length.reshape(E, 1),
                     m2_W0, m2_b0.reshape(1, H), m2_W1, m2_b1.reshape(1, H)
                     ).reshape(NW, NCH, K, HW)
    src = edge_index[0].astype(jnp.int32).reshape(NW, NCH, 1, K)
    dst = edge_index[1].astype(jnp.int32).reshape(NW, NCH, 1, K)

    conv = x
    for conv_id in range(NUM_CONVS):
        parts = _make_sc_propagate()(xw, wm_w, src, dst)
        conv, xw = _update(parts, conv,
                           m1_W0, m1_b0.reshape(1, H),
                           m1_W1, m1_b1.reshape(1, H),
                           apply_relu=conv_id < NUM_CONVS - 1)
    return conv
